# Initial kernel scaffold; baseline (speedup 1.0000x reference)
import jax
import jax.numpy as jnp
from jax.experimental import pallas as pl


def kernel(x, table):
    raise NotImplementedError("write your pallas kernel here")


# speedup vs baseline: 23.4258x; 23.4258x over previous
import functools
import jax
import jax.numpy as jnp
from jax import lax
from jax.experimental import pallas as pl
from jax.experimental.pallas import tpu as pltpu
from jax.experimental.pallas import tpu_sc as plsc

B, H, W, E, V = 256, 128, 128, 16, 256
P = H * W
NC, NS = 2, 16
NW = NC * NS
B_PER_W = B // NW
C = 2048
NCH = P // C
K = B_PER_W * NCH
L = 16

_mesh = plsc.VectorSubcoreMesh(core_axis_name="c", subcore_axis_name="s")


@functools.partial(
    pl.kernel,
    out_type=jax.ShapeDtypeStruct((B, E, P), jnp.float32),
    mesh=_mesh,
    scratch_types=[
        pltpu.VMEM((V * E,), jnp.float32),
        pltpu.VMEM((C,), jnp.int32),
        pltpu.VMEM((C,), jnp.int32),
        pltpu.VMEM((E, C), jnp.float32),
        pltpu.VMEM((E, C), jnp.float32),
        pltpu.SemaphoreType.DMA,
        pltpu.SemaphoreType.DMA,
        pltpu.SemaphoreType.DMA,
        pltpu.SemaphoreType.DMA,
    ],
    compiler_params=pltpu.CompilerParams(needs_layout_passes=False),
)
def _embed(x_hbm, table_hbm, out_hbm, tab_v, x0, x1, o0, o1,
           isem0, isem1, osem0, osem1):
    wid = lax.axis_index("s") * NC + lax.axis_index("c")
    k0_base = wid * K

    xb = (x0, x1)
    ob = (o0, o1)
    isem = (isem0, isem1)
    osem = (osem0, osem1)

    def start_in(k, buf):
        g = k0_base + k
        pltpu.async_copy(x_hbm.at[pl.ds(g * C, C)], xb[buf], isem[buf])

    def wait_in(buf):
        pltpu.make_async_copy(x_hbm.at[pl.ds(0, C)], xb[buf], isem[buf]).wait()

    def start_out(k, buf):
        g = k0_base + k
        b = g // NCH
        c0 = (g % NCH) * C
        pltpu.async_copy(ob[buf], out_hbm.at[b, :, pl.ds(c0, C)], osem[buf])

    def wait_out(buf):
        pltpu.make_async_copy(
            ob[buf], out_hbm.at[0, :, pl.ds(0, C)], osem[buf]).wait()

    def compute(buf):
        xv_ref = xb[buf]
        ov_ref = ob[buf]

        @plsc.parallel_loop(0, C, step=L, unroll=4)
        def _gathers(i):
            xv = xv_ref[pl.ds(i, L)]
            base = jnp.clip(xv, 0, V - 1) * E
            for e in range(E):
                ov_ref[e, pl.ds(i, L)] = plsc.load_gather(tab_v, [base + e])

    pltpu.sync_copy(table_hbm, tab_v)
    start_in(0, 0)
    start_in(1, 1)

    def body(j, carry):
        k = 2 * j

        wait_in(0)

        @pl.when(j > 0)
        def _():
            wait_out(0)

        compute(0)
        start_out(k, 0)

        @pl.when(j < K // 2 - 1)
        def _():
            start_in(k + 2, 0)

        wait_in(1)

        @pl.when(j > 0)
        def _():
            wait_out(1)

        compute(1)
        start_out(k + 1, 1)

        @pl.when(j < K // 2 - 1)
        def _():
            start_in(k + 3, 1)

        return carry

    lax.fori_loop(0, K // 2, body, 0, unroll=False)
    wait_out(0)
    wait_out(1)


def kernel(x, table):
    if x.ndim == 4 and x.shape[1] == 1:
        x = jnp.squeeze(x, axis=1)
    x = x.astype(jnp.int32).reshape(B * P)
    out = _embed(x, table.reshape(-1))
    return out.reshape(B, E, H, W)
